# Initial kernel scaffold; baseline (speedup 1.0000x reference)
import jax
import jax.numpy as jnp
from jax.experimental import pallas as pl


def kernel(user, item_i, item_j, embed_user, embed_item):
    raise NotImplementedError("write your pallas kernel here")


# speedup vs baseline: 1.2050x; 1.2050x over previous
import functools

import jax
import jax.numpy as jnp
from jax import lax
from jax.experimental import pallas as pl
from jax.experimental.pallas import tpu as pltpu
from jax.experimental.pallas import tpu_sc as plsc

_info = plsc.get_sparse_core_info()
_NC, _NS, _L = _info.num_cores, _info.num_subcores, _info.num_lanes
_NW = _NC * _NS


@functools.lru_cache(maxsize=None)
def _make_kernel(B, D, C):
    rows_per_w = B // _NW
    n_chunks = rows_per_w // C
    mesh = plsc.VectorSubcoreMesh(core_axis_name="c", subcore_axis_name="s")

    @functools.partial(
        pl.kernel,
        mesh=mesh,
        compiler_params=pltpu.CompilerParams(needs_layout_passes=False),
        out_type=[
            jax.ShapeDtypeStruct((B,), jnp.float32),
            jax.ShapeDtypeStruct((B,), jnp.float32),
        ],
        scratch_types=[
            pltpu.VMEM((n_chunks, C), jnp.int32),
            pltpu.VMEM((n_chunks, C), jnp.int32),
            pltpu.VMEM((n_chunks, C), jnp.int32),
            pltpu.VMEM((C, D), jnp.float32),
            pltpu.VMEM((C, D), jnp.float32),
            pltpu.VMEM((C, D), jnp.float32),
            pltpu.VMEM((C,), jnp.float32),
            pltpu.VMEM((C,), jnp.float32),
            pltpu.SemaphoreType.DMA,
        ],
    )
    def mf_kernel(user_h, item_i_h, item_j_h, eu_h, ei_h,
                  out_i_h, out_j_h,
                  uidx, iidx, jidx, eu_v, ei_v, ej_v, oi_v, oj_v, sem):
        wid = lax.axis_index("s") * _NC + lax.axis_index("c")
        base = wid * rows_per_w

        for c in range(n_chunks):
            pltpu.sync_copy(user_h.at[pl.ds(base + c * C, C)], uidx.at[c])
            pltpu.sync_copy(item_i_h.at[pl.ds(base + c * C, C)], iidx.at[c])
            pltpu.sync_copy(item_j_h.at[pl.ds(base + c * C, C)], jidx.at[c])

        for c in range(n_chunks):
            cp_u = pltpu.async_copy(eu_h.at[uidx.at[c]], eu_v, sem)
            cp_i = pltpu.async_copy(ei_h.at[iidx.at[c]], ei_v, sem)
            cp_j = pltpu.async_copy(ei_h.at[jidx.at[c]], ej_v, sem)
            cp_u.wait()
            cp_i.wait()
            cp_j.wait()

            lane = lax.iota(jnp.int32, _L)

            def group_body(g, carry):
                packed_i = jnp.zeros((_L,), jnp.float32)
                packed_j = jnp.zeros((_L,), jnp.float32)
                for k in range(_L):
                    r = g * _L + k
                    acc_i = jnp.zeros((_L,), jnp.float32)
                    acc_j = jnp.zeros((_L,), jnp.float32)
                    for s in range(D // _L):
                        eu = eu_v[r, pl.ds(s * _L, _L)]
                        acc_i = acc_i + eu * ei_v[r, pl.ds(s * _L, _L)]
                        acc_j = acc_j + eu * ej_v[r, pl.ds(s * _L, _L)]
                    packed_i = jnp.where(lane == k, jnp.sum(acc_i), packed_i)
                    packed_j = jnp.where(lane == k, jnp.sum(acc_j), packed_j)
                oi_v[pl.ds(g * _L, _L)] = packed_i
                oj_v[pl.ds(g * _L, _L)] = packed_j
                return carry

            lax.fori_loop(0, C // _L, group_body, 0)

            pltpu.sync_copy(oi_v, out_i_h.at[pl.ds(base + c * C, C)])
            pltpu.sync_copy(oj_v, out_j_h.at[pl.ds(base + c * C, C)])

    return mf_kernel

def kernel(user, item_i, item_j, embed_user, embed_item):
    B = user.shape[0]
    D = embed_user.shape[1]
    k = _make_kernel(B, D, 128)
    out_i, out_j = k(user.astype(jnp.int32), item_i.astype(jnp.int32),
                     item_j.astype(jnp.int32), embed_user, embed_item)
    return (out_i, out_j)


# speedup vs baseline: 1.3609x; 1.1294x over previous
import functools

import jax
import jax.numpy as jnp
from jax import lax
from jax.experimental import pallas as pl
from jax.experimental.pallas import tpu as pltpu
from jax.experimental.pallas import tpu_sc as plsc

_info = plsc.get_sparse_core_info()
_NC, _NS, _L = _info.num_cores, _info.num_subcores, _info.num_lanes
_NW = _NC * _NS


@functools.lru_cache(maxsize=None)
def _make_kernel(B, D, C):
    rows_per_w = B // _NW
    n_chunks = rows_per_w // C
    mesh = plsc.VectorSubcoreMesh(core_axis_name="c", subcore_axis_name="s")

    @functools.partial(
        pl.kernel,
        mesh=mesh,
        compiler_params=pltpu.CompilerParams(needs_layout_passes=False),
        out_type=[
            jax.ShapeDtypeStruct((B,), jnp.float32),
            jax.ShapeDtypeStruct((B,), jnp.float32),
        ],
        scratch_types=[
            pltpu.VMEM((n_chunks, C), jnp.int32),
            pltpu.VMEM((n_chunks, C), jnp.int32),
            pltpu.VMEM((n_chunks, C), jnp.int32),
            pltpu.VMEM((C, D), jnp.float32),
            pltpu.VMEM((C, D), jnp.float32),
            pltpu.VMEM((C, D), jnp.float32),
            pltpu.VMEM((C, D), jnp.float32),
            pltpu.VMEM((C, D), jnp.float32),
            pltpu.VMEM((C, D), jnp.float32),
            pltpu.VMEM((n_chunks, C), jnp.float32),
            pltpu.VMEM((n_chunks, C), jnp.float32),
            pltpu.SemaphoreType.DMA,
            pltpu.SemaphoreType.DMA,
            pltpu.SemaphoreType.DMA,
        ],
    )
    def mf_kernel(user_h, item_i_h, item_j_h, eu_h, ei_h,
                  out_i_h, out_j_h,
                  uidx, iidx, jidx, eu0, ei0, ej0, eu1, ei1, ej1,
                  oi_v, oj_v, gsem0, gsem1, ssem):
        wid = lax.axis_index("s") * _NC + lax.axis_index("c")
        base = wid * rows_per_w
        eu_s = (eu0, eu1)
        ei_s = (ei0, ei1)
        ej_s = (ej0, ej1)
        gsem = (gsem0, gsem1)
        lane = lax.iota(jnp.int32, _L)

        for c in range(n_chunks):
            pltpu.sync_copy(user_h.at[pl.ds(base + c * C, C)], uidx.at[c])
            pltpu.sync_copy(item_i_h.at[pl.ds(base + c * C, C)], iidx.at[c])
            pltpu.sync_copy(item_j_h.at[pl.ds(base + c * C, C)], jidx.at[c])

        def fire(c):
            slot = c % 2
            return (
                pltpu.async_copy(eu_h.at[uidx.at[c]], eu_s[slot], gsem[slot]),
                pltpu.async_copy(ei_h.at[iidx.at[c]], ei_s[slot], gsem[slot]),
                pltpu.async_copy(ei_h.at[jidx.at[c]], ej_s[slot], gsem[slot]),
            )

        inflight = {0: fire(0)}
        store_cps = []
        for c in range(n_chunks):
            slot = c % 2
            if c + 1 < n_chunks:
                inflight[c + 1] = fire(c + 1)
            for cp in inflight.pop(c):
                cp.wait()
            eu_v, ei_v, ej_v = eu_s[slot], ei_s[slot], ej_s[slot]

            def group_body(g, carry):
                packed_i = jnp.zeros((_L,), jnp.float32)
                packed_j = jnp.zeros((_L,), jnp.float32)
                for k in range(_L):
                    r = g * _L + k
                    acc_i = jnp.zeros((_L,), jnp.float32)
                    acc_j = jnp.zeros((_L,), jnp.float32)
                    for s in range(D // _L):
                        eu = eu_v[r, pl.ds(s * _L, _L)]
                        acc_i = acc_i + eu * ei_v[r, pl.ds(s * _L, _L)]
                        acc_j = acc_j + eu * ej_v[r, pl.ds(s * _L, _L)]
                    packed_i = jnp.where(lane == k, jnp.sum(acc_i), packed_i)
                    packed_j = jnp.where(lane == k, jnp.sum(acc_j), packed_j)
                oi_v[c, pl.ds(g * _L, _L)] = packed_i
                oj_v[c, pl.ds(g * _L, _L)] = packed_j
                return carry

            lax.fori_loop(0, C // _L, group_body, 0)

            store_cps.append(pltpu.async_copy(
                oi_v.at[c], out_i_h.at[pl.ds(base + c * C, C)], ssem))
            store_cps.append(pltpu.async_copy(
                oj_v.at[c], out_j_h.at[pl.ds(base + c * C, C)], ssem))

        for cp in store_cps:
            cp.wait()

    return mf_kernel

def kernel(user, item_i, item_j, embed_user, embed_item):
    B = user.shape[0]
    D = embed_user.shape[1]
    k = _make_kernel(B, D, 128)
    out_i, out_j = k(user.astype(jnp.int32), item_i.astype(jnp.int32),
                     item_j.astype(jnp.int32), embed_user, embed_item)
    return (out_i, out_j)
